# Initial kernel scaffold; baseline (speedup 1.0000x reference)
import jax
import jax.numpy as jnp
from jax.experimental import pallas as pl


def kernel(x, edge_src, edge_dst, W1, W2, W3, W4):
    raise NotImplementedError("write your pallas kernel here")


# speedup vs baseline: 2.5980x; 2.5980x over previous
import functools

import jax
import jax.numpy as jnp
from jax import lax
from jax.experimental import pallas as pl
from jax.experimental.pallas import tpu as pltpu
from jax.experimental.pallas import tpu_sc as plsc

N_NODES = 10000
C = 128
NSUB = 16
NCORE = 2
N_PAD = 10112
ROWS_PER_SUB = N_PAD // NSUB
CHUNK = 128
LANES = 16

_HIGH = jax.lax.Precision.HIGHEST

_mesh = plsc.VectorSubcoreMesh(core_axis_name="c", subcore_axis_name="s")


def _mm2_body(x_ref, w1_ref, w2_ref, h1_ref, h2_ref):
    xv = x_ref[...]
    h1_ref[...] = jnp.dot(xv, w1_ref[...], preferred_element_type=jnp.float32,
                          precision=_HIGH)
    h2_ref[...] = jnp.dot(xv, w2_ref[...], preferred_element_type=jnp.float32,
                          precision=_HIGH)


def _sig_body(a_ref, o_ref):
    o_ref[...] = jax.nn.sigmoid(a_ref[...])


def _final_body(a2_ref, s_ref, w3_ref, w4_ref, o_ref):
    a2 = a2_ref[0] + a2_ref[1]
    z = jnp.dot(a2, w3_ref[...], preferred_element_type=jnp.float32,
                precision=_HIGH)
    z = z + jnp.dot(s_ref[...], w4_ref[...], preferred_element_type=jnp.float32,
                    precision=_HIGH)
    o_ref[...] = jax.nn.sigmoid(z)


def _make_pass1(chunks_per_sub):
    @functools.partial(
        pl.kernel,
        out_type=jax.ShapeDtypeStruct((2, N_PAD, C), jnp.float32),
        mesh=_mesh,
        scratch_types=[
            pltpu.VMEM((CHUNK,), jnp.int32),
            pltpu.VMEM((CHUNK,), jnp.int32),
            pltpu.VMEM((CHUNK, C), jnp.float32),
            pltpu.VMEM((CHUNK, C), jnp.float32),
            pltpu.VMEM_SHARED((N_PAD, C), jnp.float32),
        ],
    )
    def pass1(h1_hbm, h2_hbm, esrc_hbm, edst_hbm, zeros_hbm, out_hbm,
              src_v, dst_v, buf_a, buf_b, acc):
        c = lax.axis_index("c")
        s = lax.axis_index("s")
        r0 = s * ROWS_PER_SUB
        pltpu.sync_copy(zeros_hbm.at[pl.ds(r0, ROWS_PER_SUB)],
                        acc.at[pl.ds(r0, ROWS_PER_SUB)])
        plsc.subcore_barrier()

        @pl.loop(0, chunks_per_sub)
        def _(i):
            ch = (s * chunks_per_sub + i) * CHUNK
            pltpu.sync_copy(esrc_hbm.at[pl.ds(ch, CHUNK)], src_v)
            pltpu.sync_copy(edst_hbm.at[pl.ds(ch, CHUNK)], dst_v)

            @pl.when(c == 0)
            def _():
                pltpu.sync_copy(h1_hbm.at[src_v], buf_a)
                pltpu.sync_copy(buf_a, acc.at[dst_v], add=True)
                pltpu.sync_copy(h1_hbm.at[dst_v], buf_b)
                pltpu.sync_copy(buf_b, acc.at[src_v], add=True)

            @pl.when(c == 1)
            def _():
                pltpu.sync_copy(h2_hbm.at[src_v], buf_a)
                pltpu.sync_copy(h2_hbm.at[dst_v], buf_b)

                @pl.loop(0, CHUNK)
                def _(r):
                    for l in range(0, C, LANES):
                        av = buf_a[r, pl.ds(l, LANES)]
                        bv = buf_b[r, pl.ds(l, LANES)]
                        sg = 1.0 / (1.0 + jnp.exp(av - bv))
                        buf_a[r, pl.ds(l, LANES)] = sg
                        buf_b[r, pl.ds(l, LANES)] = -sg

                pltpu.sync_copy(buf_a, acc.at[dst_v], add=True)
                pltpu.sync_copy(buf_b, acc.at[src_v], add=True)

        plsc.subcore_barrier()
        pltpu.sync_copy(acc.at[pl.ds(r0, ROWS_PER_SUB)],
                        out_hbm.at[c, pl.ds(r0, ROWS_PER_SUB)])

    return pass1


def _make_pass2(chunks_per_worker):
    @functools.partial(
        pl.kernel,
        out_type=jax.ShapeDtypeStruct((2, N_PAD, C), jnp.float32),
        mesh=_mesh,
        scratch_types=[
            pltpu.VMEM((CHUNK,), jnp.int32),
            pltpu.VMEM((CHUNK,), jnp.int32),
            pltpu.VMEM((CHUNK, C), jnp.float32),
            pltpu.VMEM((CHUNK, C), jnp.float32),
            pltpu.VMEM_SHARED((N_PAD, C), jnp.float32),
        ],
    )
    def pass2(n1_hbm, esrc_hbm, edst_hbm, zeros_hbm, out_hbm,
              src_v, dst_v, buf_a, buf_b, acc):
        c = lax.axis_index("c")
        s = lax.axis_index("s")
        r0 = s * ROWS_PER_SUB
        pltpu.sync_copy(zeros_hbm.at[pl.ds(r0, ROWS_PER_SUB)],
                        acc.at[pl.ds(r0, ROWS_PER_SUB)])
        plsc.subcore_barrier()

        @pl.loop(0, chunks_per_worker)
        def _(i):
            ch = ((c * NSUB + s) * chunks_per_worker + i) * CHUNK
            pltpu.sync_copy(esrc_hbm.at[pl.ds(ch, CHUNK)], src_v)
            pltpu.sync_copy(edst_hbm.at[pl.ds(ch, CHUNK)], dst_v)
            pltpu.sync_copy(n1_hbm.at[src_v], buf_a)
            pltpu.sync_copy(buf_a, acc.at[dst_v], add=True)
            pltpu.sync_copy(n1_hbm.at[dst_v], buf_b)
            pltpu.sync_copy(buf_b, acc.at[src_v], add=True)

        plsc.subcore_barrier()
        pltpu.sync_copy(acc.at[pl.ds(r0, ROWS_PER_SUB)],
                        out_hbm.at[c, pl.ds(r0, ROWS_PER_SUB)])

    return pass2


def kernel(x, edge_src, edge_dst, W1, W2, W3, W4):
    x = x.astype(jnp.float32)
    esrc = edge_src.astype(jnp.int32)
    edst = edge_dst.astype(jnp.int32)
    n_edges = esrc.shape[0]

    n_chunks = -(-n_edges // CHUNK)
    n_chunks = -(-n_chunks // (NCORE * NSUB)) * (NCORE * NSUB)
    e_pad = n_chunks * CHUNK

    x_pad = jnp.zeros((N_PAD, C), jnp.float32).at[:N_NODES].set(x)
    pad = jnp.full((e_pad - n_edges,), N_NODES, jnp.int32)
    esrc_p = jnp.concatenate([esrc, pad])
    edst_p = jnp.concatenate([edst, pad])
    zeros = jnp.zeros((N_PAD, C), jnp.float32)

    h1, h2 = pl.pallas_call(
        _mm2_body,
        out_shape=(jax.ShapeDtypeStruct((N_PAD, C), jnp.float32),
                   jax.ShapeDtypeStruct((N_PAD, C), jnp.float32)),
    )(x_pad, W1, W2)

    p1 = _make_pass1(n_chunks // NSUB)(h1, h2, esrc_p, edst_p, zeros)
    a1, s_acc = p1[0], p1[1]

    n1 = pl.pallas_call(
        _sig_body,
        out_shape=jax.ShapeDtypeStruct((N_PAD, C), jnp.float32),
    )(a1)

    a2 = _make_pass2(n_chunks // (NCORE * NSUB))(n1, esrc_p, edst_p, zeros)

    out = pl.pallas_call(
        _final_body,
        out_shape=jax.ShapeDtypeStruct((N_PAD, C), jnp.float32),
    )(a2, s_acc, W3, W4)

    return out[:N_NODES]


# speedup vs baseline: 2.9211x; 1.1243x over previous
import functools

import jax
import jax.numpy as jnp
from jax import lax
from jax.experimental import pallas as pl
from jax.experimental.pallas import tpu as pltpu
from jax.experimental.pallas import tpu_sc as plsc

N_NODES = 10000
C = 128
NSUB = 16
NCORE = 2
N_PAD = 10112
ROWS_PER_SUB = N_PAD // NSUB
CHUNK = 64
WIN = 32
LANES = 16

_HIGH = jax.lax.Precision.HIGHEST

_mesh = plsc.VectorSubcoreMesh(core_axis_name="c", subcore_axis_name="s")


def _mm2_body(x_ref, w1_ref, w2_ref, h1_ref, h2_ref):
    xv = x_ref[...]
    h1_ref[...] = jnp.dot(xv, w1_ref[...], preferred_element_type=jnp.float32,
                          precision=_HIGH)
    h2_ref[...] = jnp.dot(xv, w2_ref[...], preferred_element_type=jnp.float32,
                          precision=_HIGH)


def _sig_body(a_ref, o_ref):
    o_ref[...] = jax.nn.sigmoid(a_ref[...])


def _final_body(a2_ref, s_ref, w3_ref, w4_ref, o_ref):
    a2 = a2_ref[0] + a2_ref[1]
    z = jnp.dot(a2, w3_ref[...], preferred_element_type=jnp.float32,
                precision=_HIGH)
    z = z + jnp.dot(s_ref[...], w4_ref[...], preferred_element_type=jnp.float32,
                    precision=_HIGH)
    o_ref[...] = jax.nn.sigmoid(z)


def _window_loop(ring, esrc_hbm, edst_hbm, ch0, nch, srcw, dstw, wsem):
    nwin = nch // WIN
    pltpu.sync_copy(esrc_hbm.at[pl.ds(ch0, WIN)], srcw.at[0])
    pltpu.sync_copy(edst_hbm.at[pl.ds(ch0, WIN)], dstw.at[0])

    @pl.loop(0, nwin)
    def _(w):
        slot = lax.rem(w, 2)
        nslot = lax.rem(w + 1, 2)

        @pl.when(w + 1 < nwin)
        def _():
            nb = ch0 + (w + 1) * WIN
            pltpu.async_copy(esrc_hbm.at[pl.ds(nb, WIN)], srcw.at[nslot], wsem)
            pltpu.async_copy(edst_hbm.at[pl.ds(nb, WIN)], dstw.at[nslot], wsem)

        ring(srcw.at[slot], dstw.at[slot])

        @pl.when(w + 1 < nwin)
        def _():
            pltpu.make_async_copy(esrc_hbm.at[pl.ds(ch0, WIN)], srcw.at[nslot],
                                  wsem).wait()
            pltpu.make_async_copy(edst_hbm.at[pl.ds(ch0, WIN)], dstw.at[nslot],
                                  wsem).wait()


def _adj_ring(h_hbm, acc, src_slab, dst_slab, bufs, gsems, ssems, nch):
    def unit(j, b):
        ci = 2 * j + (b // 2)
        gslab = src_slab if (b % 2) == 0 else dst_slab
        sslab = dst_slab if (b % 2) == 0 else src_slab
        return ci, gslab, sslab

    for b in range(4):
        ci, gslab, _ = unit(0, b)
        pltpu.async_copy(h_hbm.at[gslab.at[ci]], bufs[b], gsems[b])

    @pl.loop(0, nch // 2)
    def _(j):
        scats = []
        for b in range(4):
            ci, gslab, sslab = unit(j, b)
            pltpu.make_async_copy(h_hbm.at[gslab.at[ci]], bufs[b],
                                  gsems[b]).wait()
            scats.append(pltpu.async_copy(bufs[b], acc.at[sslab.at[ci]],
                                          ssems[b], add=True))
        for b in range(4):
            scats[b].wait()
            ci_n, gslab, _ = unit(j + 1, b)

            @pl.when(ci_n < nch)
            def _():
                pltpu.async_copy(h_hbm.at[gslab.at[ci_n]], bufs[b], gsems[b])


def _sig_ring(h_hbm, acc, src_slab, dst_slab, bufs, gsems, ssems, nch):
    def pair_gathers(ci, p):
        ga = pltpu.async_copy(h_hbm.at[src_slab.at[ci]], bufs[p], gsems[p])
        gb = pltpu.async_copy(h_hbm.at[dst_slab.at[ci]], bufs[2 + p],
                              gsems[2 + p])
        return ga, gb

    for p in range(2):
        pair_gathers(p, p)

    @pl.loop(0, nch // 2)
    def _(j):
        scats = []
        for p in range(2):
            ci = 2 * j + p
            pltpu.make_async_copy(h_hbm.at[src_slab.at[ci]], bufs[p],
                                  gsems[p]).wait()
            pltpu.make_async_copy(h_hbm.at[dst_slab.at[ci]], bufs[2 + p],
                                  gsems[2 + p]).wait()

            @pl.loop(0, CHUNK)
            def _(r):
                for l in range(0, C, LANES):
                    av = bufs[p][r, pl.ds(l, LANES)]
                    bv = bufs[2 + p][r, pl.ds(l, LANES)]
                    sg = 1.0 / (1.0 + jnp.exp(av - bv))
                    bufs[p][r, pl.ds(l, LANES)] = sg
                    bufs[2 + p][r, pl.ds(l, LANES)] = -sg

            scats.append(pltpu.async_copy(bufs[p], acc.at[dst_slab.at[ci]],
                                          ssems[p], add=True))
            scats.append(pltpu.async_copy(bufs[2 + p], acc.at[src_slab.at[ci]],
                                          ssems[2 + p], add=True))
        for p in range(2):
            ci_n = 2 * j + 2 + p
            scats[2 * p].wait()
            scats[2 * p + 1].wait()

            @pl.when(ci_n < nch)
            def _():
                pair_gathers(ci_n, p)


_SC_SCRATCH = [
    pltpu.VMEM((2, WIN, CHUNK), jnp.int32),
    pltpu.VMEM((2, WIN, CHUNK), jnp.int32),
    pltpu.VMEM((CHUNK, C), jnp.float32),
    pltpu.VMEM((CHUNK, C), jnp.float32),
    pltpu.VMEM((CHUNK, C), jnp.float32),
    pltpu.VMEM((CHUNK, C), jnp.float32),
    pltpu.VMEM_SHARED((N_PAD, C), jnp.float32),
] + [pltpu.SemaphoreType.DMA] * 9


def _make_pass1(chunks_per_sub):
    @functools.partial(
        pl.kernel,
        out_type=jax.ShapeDtypeStruct((2, N_PAD, C), jnp.float32),
        mesh=_mesh,
        scratch_types=_SC_SCRATCH,
    )
    def pass1(h1_hbm, h2_hbm, esrc_hbm, edst_hbm, zeros_hbm, out_hbm,
              srcw, dstw, b0, b1, b2, b3, acc,
              g0, g1, g2, g3, s0, s1, s2, s3, wsem):
        c = lax.axis_index("c")
        s = lax.axis_index("s")
        r0 = s * ROWS_PER_SUB
        ch0 = s * chunks_per_sub
        pltpu.sync_copy(zeros_hbm.at[pl.ds(r0, ROWS_PER_SUB)],
                        acc.at[pl.ds(r0, ROWS_PER_SUB)])
        plsc.subcore_barrier()

        bufs = (b0, b1, b2, b3)
        gsems = (g0, g1, g2, g3)
        ssems = (s0, s1, s2, s3)

        @pl.when(c == 0)
        def _():
            _window_loop(
                lambda sr, dr: _adj_ring(h1_hbm, acc, sr, dr, bufs, gsems,
                                         ssems, WIN),
                esrc_hbm, edst_hbm, ch0, chunks_per_sub, srcw, dstw, wsem)

        @pl.when(c == 1)
        def _():
            _window_loop(
                lambda sr, dr: _sig_ring(h2_hbm, acc, sr, dr, bufs, gsems,
                                         ssems, WIN),
                esrc_hbm, edst_hbm, ch0, chunks_per_sub, srcw, dstw, wsem)

        plsc.subcore_barrier()
        pltpu.sync_copy(acc.at[pl.ds(r0, ROWS_PER_SUB)],
                        out_hbm.at[c, pl.ds(r0, ROWS_PER_SUB)])

    return pass1


def _make_pass2(chunks_per_worker):
    @functools.partial(
        pl.kernel,
        out_type=jax.ShapeDtypeStruct((2, N_PAD, C), jnp.float32),
        mesh=_mesh,
        scratch_types=_SC_SCRATCH,
    )
    def pass2(n1_hbm, esrc_hbm, edst_hbm, zeros_hbm, out_hbm,
              srcw, dstw, b0, b1, b2, b3, acc,
              g0, g1, g2, g3, s0, s1, s2, s3, wsem):
        c = lax.axis_index("c")
        s = lax.axis_index("s")
        r0 = s * ROWS_PER_SUB
        ch0 = (c * NSUB + s) * chunks_per_worker
        pltpu.sync_copy(zeros_hbm.at[pl.ds(r0, ROWS_PER_SUB)],
                        acc.at[pl.ds(r0, ROWS_PER_SUB)])
        plsc.subcore_barrier()

        _window_loop(
            lambda sr, dr: _adj_ring(n1_hbm, acc, sr, dr, (b0, b1, b2, b3),
                                     (g0, g1, g2, g3), (s0, s1, s2, s3), WIN),
            esrc_hbm, edst_hbm, ch0, chunks_per_worker, srcw, dstw, wsem)

        plsc.subcore_barrier()
        pltpu.sync_copy(acc.at[pl.ds(r0, ROWS_PER_SUB)],
                        out_hbm.at[c, pl.ds(r0, ROWS_PER_SUB)])

    return pass2


def kernel(x, edge_src, edge_dst, W1, W2, W3, W4):
    x = x.astype(jnp.float32)
    esrc = edge_src.astype(jnp.int32)
    edst = edge_dst.astype(jnp.int32)
    n_edges = esrc.shape[0]

    n_chunks = -(-n_edges // CHUNK)
    align = WIN * NCORE * NSUB
    n_chunks = -(-n_chunks // align) * align
    e_pad = n_chunks * CHUNK

    x_pad = jnp.zeros((N_PAD, C), jnp.float32).at[:N_NODES].set(x)
    pad = jnp.full((e_pad - n_edges,), N_NODES, jnp.int32)
    esrc_p = jnp.concatenate([esrc, pad]).reshape(n_chunks, CHUNK)
    edst_p = jnp.concatenate([edst, pad]).reshape(n_chunks, CHUNK)
    zeros = jnp.zeros((N_PAD, C), jnp.float32)

    h1, h2 = pl.pallas_call(
        _mm2_body,
        out_shape=(jax.ShapeDtypeStruct((N_PAD, C), jnp.float32),
                   jax.ShapeDtypeStruct((N_PAD, C), jnp.float32)),
    )(x_pad, W1, W2)

    p1 = _make_pass1(n_chunks // NSUB)(h1, h2, esrc_p, edst_p, zeros)
    a1, s_acc = p1[0], p1[1]

    n1 = pl.pallas_call(
        _sig_body,
        out_shape=jax.ShapeDtypeStruct((N_PAD, C), jnp.float32),
    )(a1)

    a2 = _make_pass2(n_chunks // (NCORE * NSUB))(n1, esrc_p, edst_p, zeros)

    out = pl.pallas_call(
        _final_body,
        out_shape=jax.ShapeDtypeStruct((N_PAD, C), jnp.float32),
    )(a2, s_acc, W3, W4)

    return out[:N_NODES]


# speedup vs baseline: 6.6021x; 2.2602x over previous
import functools

import jax
import jax.numpy as jnp
from jax import lax
from jax.experimental import pallas as pl
from jax.experimental.pallas import tpu as pltpu
from jax.experimental.pallas import tpu_sc as plsc

N_NODES = 10000
C = 128
NSUB = 16
NCORE = 2
N_PAD = 10112
ROWS_PER_SUB = N_PAD // NSUB
CHUNK = 64
WIN = 32
LANES = 16

_HIGH = jax.lax.Precision.HIGHEST

_mesh = plsc.VectorSubcoreMesh(core_axis_name="c", subcore_axis_name="s")


def _mm2_body(x_ref, w1_ref, w2_ref, h1_ref, h2_ref):
    xv = x_ref[...]
    h1_ref[...] = jnp.dot(xv, w1_ref[...], preferred_element_type=jnp.float32,
                          precision=_HIGH)
    h2_ref[...] = jnp.dot(xv, w2_ref[...], preferred_element_type=jnp.float32,
                          precision=_HIGH)


def _sig_body(a_ref, o_ref):
    o_ref[...] = jax.nn.sigmoid(a_ref[...])


def _final_body(a2_ref, s_ref, w3_ref, w4_ref, o_ref):
    a2 = a2_ref[0] + a2_ref[1]
    z = jnp.dot(a2, w3_ref[...], preferred_element_type=jnp.float32,
                precision=_HIGH)
    z = z + jnp.dot(s_ref[...], w4_ref[...], preferred_element_type=jnp.float32,
                    precision=_HIGH)
    o_ref[...] = jax.nn.sigmoid(z)


def _window_loop(ring, esrc_hbm, edst_hbm, ch0, nch, srcw, dstw, wsem):
    nwin = nch // WIN
    pltpu.sync_copy(esrc_hbm.at[pl.ds(ch0, WIN)], srcw.at[0])
    pltpu.sync_copy(edst_hbm.at[pl.ds(ch0, WIN)], dstw.at[0])

    @pl.loop(0, nwin)
    def _(w):
        slot = lax.rem(w, 2)
        nslot = lax.rem(w + 1, 2)

        @pl.when(w + 1 < nwin)
        def _():
            nb = ch0 + (w + 1) * WIN
            pltpu.async_copy(esrc_hbm.at[pl.ds(nb, WIN)], srcw.at[nslot], wsem)
            pltpu.async_copy(edst_hbm.at[pl.ds(nb, WIN)], dstw.at[nslot], wsem)

        ring(srcw.at[slot], dstw.at[slot])

        @pl.when(w + 1 < nwin)
        def _():
            pltpu.make_async_copy(esrc_hbm.at[pl.ds(ch0, WIN)], srcw.at[nslot],
                                  wsem).wait()
            pltpu.make_async_copy(edst_hbm.at[pl.ds(ch0, WIN)], dstw.at[nslot],
                                  wsem).wait()


def _adj_ring(h_hbm, acc, src_slab, dst_slab, bufs, gsems, ssems, nch):
    def unit(j, b):
        ci = 2 * j + (b // 2)
        gslab = src_slab if (b % 2) == 0 else dst_slab
        sslab = dst_slab if (b % 2) == 0 else src_slab
        return ci, gslab, sslab

    for b in range(4):
        ci, gslab, _ = unit(0, b)
        pltpu.async_copy(h_hbm.at[gslab.at[ci]], bufs[b], gsems[b])

    @pl.loop(0, nch // 2)
    def _(j):
        scats = []
        for b in range(4):
            ci, gslab, sslab = unit(j, b)
            pltpu.make_async_copy(h_hbm.at[gslab.at[ci]], bufs[b],
                                  gsems[b]).wait()
            scats.append(pltpu.async_copy(bufs[b], acc.at[sslab.at[ci]],
                                          ssems[b], add=True))
        for b in range(4):
            scats[b].wait()
            ci_n, gslab, _ = unit(j + 1, b)

            @pl.when(ci_n < nch)
            def _():
                pltpu.async_copy(h_hbm.at[gslab.at[ci_n]], bufs[b], gsems[b])


def _sig_ring(h_hbm, acc, src_slab, dst_slab, bufs, gsems, ssems, nch):
    def pair_gathers(ci, p):
        ga = pltpu.async_copy(h_hbm.at[src_slab.at[ci]], bufs[p], gsems[p])
        gb = pltpu.async_copy(h_hbm.at[dst_slab.at[ci]], bufs[2 + p],
                              gsems[2 + p])
        return ga, gb

    for p in range(2):
        pair_gathers(p, p)

    @pl.loop(0, nch // 2)
    def _(j):
        scats = []
        for p in range(2):
            ci = 2 * j + p
            pltpu.make_async_copy(h_hbm.at[src_slab.at[ci]], bufs[p],
                                  gsems[p]).wait()
            pltpu.make_async_copy(h_hbm.at[dst_slab.at[ci]], bufs[2 + p],
                                  gsems[2 + p]).wait()

            @pl.loop(0, CHUNK)
            def _(r):
                for l in range(0, C, LANES):
                    av = bufs[p][r, pl.ds(l, LANES)]
                    bv = bufs[2 + p][r, pl.ds(l, LANES)]
                    sg = 1.0 / (1.0 + jnp.exp(av - bv))
                    bufs[p][r, pl.ds(l, LANES)] = sg
                    bufs[2 + p][r, pl.ds(l, LANES)] = -sg

            scats.append(pltpu.async_copy(bufs[p], acc.at[dst_slab.at[ci]],
                                          ssems[p], add=True))
            scats.append(pltpu.async_copy(bufs[2 + p], acc.at[src_slab.at[ci]],
                                          ssems[2 + p], add=True))
        for p in range(2):
            ci_n = 2 * j + 2 + p
            scats[2 * p].wait()
            scats[2 * p + 1].wait()

            @pl.when(ci_n < nch)
            def _():
                pair_gathers(ci_n, p)


_SC_SCRATCH = [
    pltpu.VMEM((2, WIN, CHUNK), jnp.int32),
    pltpu.VMEM((2, WIN, CHUNK), jnp.int32),
    pltpu.VMEM((CHUNK, C), jnp.float32),
    pltpu.VMEM((CHUNK, C), jnp.float32),
    pltpu.VMEM((CHUNK, C), jnp.float32),
    pltpu.VMEM((CHUNK, C), jnp.float32),
    pltpu.VMEM_SHARED((N_PAD, C), jnp.float32),
] + [pltpu.SemaphoreType.DMA] * 9


def _make_pass1(chunks_per_sub):
    @functools.partial(
        pl.kernel,
        out_type=jax.ShapeDtypeStruct((2, N_PAD, C), jnp.float32),
        mesh=_mesh,
        scratch_types=_SC_SCRATCH,
    )
    def pass1(h1_hbm, h2_hbm, esrc_hbm, edst_hbm, zeros_hbm, out_hbm,
              srcw, dstw, b0, b1, b2, b3, acc,
              g0, g1, g2, g3, s0, s1, s2, s3, wsem):
        c = lax.axis_index("c")
        s = lax.axis_index("s")
        r0 = s * ROWS_PER_SUB
        ch0 = s * chunks_per_sub
        pltpu.sync_copy(zeros_hbm.at[pl.ds(r0, ROWS_PER_SUB)],
                        acc.at[pl.ds(r0, ROWS_PER_SUB)])
        plsc.subcore_barrier()

        bufs = (b0, b1, b2, b3)
        gsems = (g0, g1, g2, g3)
        ssems = (s0, s1, s2, s3)

        @pl.when(c == 0)
        def _():
            _window_loop(
                lambda sr, dr: _adj_ring(h1_hbm, acc, sr, dr, bufs, gsems,
                                         ssems, WIN),
                esrc_hbm, edst_hbm, ch0, chunks_per_sub, srcw, dstw, wsem)

        @pl.when(c == 1)
        def _():
            _window_loop(
                lambda sr, dr: _sig_ring(h2_hbm, acc, sr, dr, bufs, gsems,
                                         ssems, WIN),
                esrc_hbm, edst_hbm, ch0, chunks_per_sub, srcw, dstw, wsem)

        plsc.subcore_barrier()
        pltpu.sync_copy(acc.at[pl.ds(r0, ROWS_PER_SUB)],
                        out_hbm.at[c, pl.ds(r0, ROWS_PER_SUB)])

    return pass1


def _make_pass2(chunks_per_worker):
    @functools.partial(
        pl.kernel,
        out_type=jax.ShapeDtypeStruct((2, N_PAD, C), jnp.float32),
        mesh=_mesh,
        scratch_types=_SC_SCRATCH,
    )
    def pass2(n1_hbm, esrc_hbm, edst_hbm, zeros_hbm, out_hbm,
              srcw, dstw, b0, b1, b2, b3, acc,
              g0, g1, g2, g3, s0, s1, s2, s3, wsem):
        c = lax.axis_index("c")
        s = lax.axis_index("s")
        r0 = s * ROWS_PER_SUB
        ch0 = (c * NSUB + s) * chunks_per_worker
        pltpu.sync_copy(zeros_hbm.at[pl.ds(r0, ROWS_PER_SUB)],
                        acc.at[pl.ds(r0, ROWS_PER_SUB)])
        plsc.subcore_barrier()

        _window_loop(
            lambda sr, dr: _adj_ring(n1_hbm, acc, sr, dr, (b0, b1, b2, b3),
                                     (g0, g1, g2, g3), (s0, s1, s2, s3), WIN),
            esrc_hbm, edst_hbm, ch0, chunks_per_worker, srcw, dstw, wsem)

        plsc.subcore_barrier()
        pltpu.sync_copy(acc.at[pl.ds(r0, ROWS_PER_SUB)],
                        out_hbm.at[c, pl.ds(r0, ROWS_PER_SUB)])

    return pass2


def kernel(x, edge_src, edge_dst, W1, W2, W3, W4):
    x = x.astype(jnp.float32)
    esrc = edge_src.astype(jnp.int32)
    edst = edge_dst.astype(jnp.int32)
    n_edges = esrc.shape[0]

    n_chunks = -(-n_edges // CHUNK)
    align = WIN * NCORE * NSUB
    n_chunks = -(-n_chunks // align) * align
    e_pad = n_chunks * CHUNK

    x_pad = jnp.zeros((N_PAD, C), jnp.float32).at[:N_NODES].set(x)
    pad = N_NODES + (jnp.arange(e_pad - n_edges, dtype=jnp.int32)
                     % (N_PAD - N_NODES))
    esrc_p = jnp.concatenate([esrc, pad]).reshape(n_chunks, CHUNK)
    edst_p = jnp.concatenate([edst, pad]).reshape(n_chunks, CHUNK)
    zeros = jnp.zeros((N_PAD, C), jnp.float32)

    h1, h2 = pl.pallas_call(
        _mm2_body,
        out_shape=(jax.ShapeDtypeStruct((N_PAD, C), jnp.float32),
                   jax.ShapeDtypeStruct((N_PAD, C), jnp.float32)),
    )(x_pad, W1, W2)

    p1 = _make_pass1(n_chunks // NSUB)(h1, h2, esrc_p, edst_p, zeros)
    a1, s_acc = p1[0], p1[1]

    n1 = pl.pallas_call(
        _sig_body,
        out_shape=jax.ShapeDtypeStruct((N_PAD, C), jnp.float32),
    )(a1)

    a2 = _make_pass2(n_chunks // (NCORE * NSUB))(n1, esrc_p, edst_p, zeros)

    out = pl.pallas_call(
        _final_body,
        out_shape=jax.ShapeDtypeStruct((N_PAD, C), jnp.float32),
    )(a2, s_acc, W3, W4)

    return out[:N_NODES]
